# Initial kernel scaffold; baseline (speedup 1.0000x reference)
import jax
import jax.numpy as jnp
from jax.experimental import pallas as pl


def kernel(bpe_ids, phoneme_ids, bpe_table, ph_table, W_bpe, W_ph, W_text):
    raise NotImplementedError("write your pallas kernel here")


# speedup vs baseline: 2.1298x; 2.1298x over previous
import functools

import jax
import jax.numpy as jnp
from jax import lax
from jax.experimental import pallas as pl
from jax.experimental.pallas import tpu as pltpu
from jax.experimental.pallas import tpu_sc as plsc

B = 8
L_BPE = 2048
BPE_PER_WORD = 4
PH_PER_WORD = 2
N_WORDS = L_BPE // BPE_PER_WORD
NB = B * N_WORDS
L_PH = N_WORDS * PH_PER_WORD
D = 256
V_PH_PAD = 128

NC = 2
NS = 16
NW = NC * NS
ROWS = BPE_PER_WORD * NB
ROWS_PER_W = ROWS // NW
CHUNK = 128
NCHUNK = ROWS_PER_W // CHUNK

WB = 512
GRID = NB // WB


def _sc_gather(idx, table):
    mesh = plsc.VectorSubcoreMesh(core_axis_name="c", subcore_axis_name="s")

    @functools.partial(
        pl.kernel,
        mesh=mesh,
        out_type=jax.ShapeDtypeStruct((ROWS, D), jnp.float32),
        scratch_types=[
            pltpu.VMEM((NCHUNK, CHUNK), jnp.int32),
            pltpu.VMEM((CHUNK, D), jnp.float32),
            pltpu.VMEM((CHUNK, D), jnp.float32),
            pltpu.VMEM((CHUNK, D), jnp.float32),
            pltpu.SemaphoreType.DMA,
            pltpu.SemaphoreType.DMA,
            pltpu.SemaphoreType.DMA,
            pltpu.SemaphoreType.DMA,
            pltpu.SemaphoreType.DMA,
            pltpu.SemaphoreType.DMA,
        ],
    )
    def k(idx_hbm, table_hbm, out_hbm, idx_v, buf0, buf1, buf2,
          gs0, gs1, gs2, os0, os1, os2):
        wid = lax.axis_index("s") * NC + lax.axis_index("c")
        base = wid * ROWS_PER_W
        pltpu.sync_copy(idx_hbm.at[wid], idx_v)
        bufs = (buf0, buf1, buf2)
        gsems = (gs0, gs1, gs2)
        osems = (os0, os1, os2)
        gathers = [None] * NCHUNK
        outs = [None] * NCHUNK
        gathers[0] = pltpu.async_copy(table_hbm.at[idx_v.at[0]], bufs[0], gsems[0])
        if NCHUNK > 1:
            gathers[1] = pltpu.async_copy(table_hbm.at[idx_v.at[1]], bufs[1], gsems[1])
        for c in range(NCHUNK):
            gathers[c].wait()
            if c >= 1:
                outs[c - 1].wait()
            outs[c] = pltpu.async_copy(
                bufs[c % 3], out_hbm.at[pl.ds(base + c * CHUNK, CHUNK)], osems[c % 3]
            )
            if c + 2 < NCHUNK:
                gathers[c + 2] = pltpu.async_copy(
                    table_hbm.at[idx_v.at[c + 2]], bufs[(c + 2) % 3], gsems[(c + 2) % 3]
                )
        outs[NCHUNK - 1].wait()

    return k(idx, table)


def _gelu(x):
    u = 0.7978845608028654 * (x + 0.044715 * x * x * x)
    t = jnp.exp(-2.0 * jnp.abs(u))
    tanh_u = jnp.sign(u) * (1.0 - t) / (1.0 + t)
    return 0.5 * x * (1.0 + tanh_u)


def _tc_body(g_ref, id0_ref, id1_ref, pht_ref, wph_ref, wbpe_ref, wtext_ref,
             out_ref, phproj_ref):
    @pl.when(pl.program_id(0) == 0)
    def _():
        phproj_ref[...] = _gelu(
            jnp.dot(pht_ref[...], wph_ref[...], preferred_element_type=jnp.float32, precision=lax.Precision.HIGHEST)
        )

    wbpe = wbpe_ref[...]
    acc = _gelu(jnp.dot(g_ref[0], wbpe, preferred_element_type=jnp.float32, precision=lax.Precision.HIGHEST))
    for j in range(1, BPE_PER_WORD):
        acc = acc + _gelu(
            jnp.dot(g_ref[j], wbpe, preferred_element_type=jnp.float32, precision=lax.Precision.HIGHEST)
        )
    bpe_mean = acc * (1.0 / BPE_PER_WORD)

    iota = lax.broadcasted_iota(jnp.int32, (WB, V_PH_PAD), 1)
    phproj = phproj_ref[...]
    wtext = wtext_ref[...]
    out = None
    for id_ref in (id0_ref, id1_ref):
        oh = (id_ref[...] == iota).astype(jnp.float32)
        v = jnp.dot(oh, phproj, preferred_element_type=jnp.float32, precision=lax.Precision.HIGHEST)
        t = _gelu(
            jnp.dot(v + bpe_mean, wtext, preferred_element_type=jnp.float32, precision=lax.Precision.HIGHEST)
        )
        out = t if out is None else out + t
    out_ref[...] = out * (1.0 / PH_PER_WORD)


def _tc_fused(gathered, ids0, ids1, ph_table_pad, W_ph, W_bpe, W_text):
    full = lambda shape: pl.BlockSpec(shape, lambda i: tuple(0 for _ in shape))
    return pl.pallas_call(
        _tc_body,
        grid=(GRID,),
        in_specs=[
            pl.BlockSpec((BPE_PER_WORD, WB, D), lambda i: (0, i, 0)),
            pl.BlockSpec((WB, V_PH_PAD), lambda i: (i, 0)),
            pl.BlockSpec((WB, V_PH_PAD), lambda i: (i, 0)),
            full((V_PH_PAD, D)),
            full((D, D)),
            full((D, D)),
            full((D, D)),
        ],
        out_specs=pl.BlockSpec((WB, D), lambda i: (i, 0)),
        out_shape=jax.ShapeDtypeStruct((NB, D), jnp.float32),
        scratch_shapes=[pltpu.VMEM((V_PH_PAD, D), jnp.float32)],
    )(gathered, ids0, ids1, ph_table_pad, W_ph, W_bpe, W_text)


def kernel(bpe_ids, phoneme_ids, bpe_table, ph_table, W_bpe, W_ph, W_text):
    idx = bpe_ids.reshape(B, N_WORDS, BPE_PER_WORD).transpose(2, 0, 1)
    idx = idx.reshape(NW, NCHUNK, CHUNK)

    gathered = _sc_gather(idx, bpe_table)
    gathered = gathered.reshape(BPE_PER_WORD, NB, D)

    ph = phoneme_ids.reshape(B, N_WORDS, PH_PER_WORD)
    ids0 = jnp.broadcast_to(ph[:, :, 0].reshape(NB, 1), (NB, V_PH_PAD))
    ids1 = jnp.broadcast_to(ph[:, :, 1].reshape(NB, 1), (NB, V_PH_PAD))

    ph_table_pad = jnp.zeros((V_PH_PAD, D), jnp.float32).at[:ph_table.shape[0]].set(ph_table)

    return _tc_fused(gathered, ids0, ids1, ph_table_pad, W_ph, W_bpe, W_text)


# speedup vs baseline: 3.5258x; 1.6554x over previous
import functools

import jax
import jax.numpy as jnp
from jax import lax
from jax.experimental import pallas as pl
from jax.experimental.pallas import tpu as pltpu
from jax.experimental.pallas import tpu_sc as plsc

B = 8
L_BPE = 2048
BPE_PER_WORD = 4
PH_PER_WORD = 2
N_WORDS = L_BPE // BPE_PER_WORD
NB = B * N_WORDS
L_PH = N_WORDS * PH_PER_WORD
D = 256
V_PH_PAD = 128

NC = 2
NS = 16
NW = NC * NS
ROWS = BPE_PER_WORD * NB
ROWS_PER_W = ROWS // NW
CHUNK = 128
NCHUNK = ROWS_PER_W // CHUNK

WB = 512
GRID = NB // WB


def _sc_gather(idx, table):
    mesh = plsc.VectorSubcoreMesh(core_axis_name="c", subcore_axis_name="s")

    @functools.partial(
        pl.kernel,
        mesh=mesh,
        out_type=jax.ShapeDtypeStruct((ROWS, D), jnp.float32),
        scratch_types=[
            pltpu.VMEM((NCHUNK, CHUNK), jnp.int32),
            pltpu.VMEM((CHUNK, D), jnp.float32),
            pltpu.VMEM((CHUNK, D), jnp.float32),
            pltpu.VMEM((CHUNK, D), jnp.float32),
            pltpu.SemaphoreType.DMA,
            pltpu.SemaphoreType.DMA,
            pltpu.SemaphoreType.DMA,
            pltpu.SemaphoreType.DMA,
            pltpu.SemaphoreType.DMA,
            pltpu.SemaphoreType.DMA,
        ],
    )
    def k(idx_hbm, table_hbm, out_hbm, idx_v, buf0, buf1, buf2,
          gs0, gs1, gs2, os0, os1, os2):
        wid = lax.axis_index("s") * NC + lax.axis_index("c")
        base = wid * ROWS_PER_W
        pltpu.sync_copy(idx_hbm.at[wid], idx_v)
        bufs = (buf0, buf1, buf2)
        gsems = (gs0, gs1, gs2)
        osems = (os0, os1, os2)
        gathers = [None] * NCHUNK
        outs = [None] * NCHUNK
        gathers[0] = pltpu.async_copy(table_hbm.at[idx_v.at[0]], bufs[0], gsems[0])
        if NCHUNK > 1:
            gathers[1] = pltpu.async_copy(table_hbm.at[idx_v.at[1]], bufs[1], gsems[1])
        for c in range(NCHUNK):
            gathers[c].wait()
            if c >= 1:
                outs[c - 1].wait()
            outs[c] = pltpu.async_copy(
                bufs[c % 3], out_hbm.at[pl.ds(base + c * CHUNK, CHUNK)], osems[c % 3]
            )
            if c + 2 < NCHUNK:
                gathers[c + 2] = pltpu.async_copy(
                    table_hbm.at[idx_v.at[c + 2]], bufs[(c + 2) % 3], gsems[(c + 2) % 3]
                )
        outs[NCHUNK - 1].wait()

    return k(idx, table)


_PREC = lax.Precision.DEFAULT


def _dot(a, b):
    return jnp.dot(a, b, preferred_element_type=jnp.float32, precision=_PREC)


def _gelu(x):
    u = 0.7978845608028654 * (x + 0.044715 * x * x * x)
    return x / (1.0 + jnp.exp(-2.0 * u))


def _tc_body(g_ref, id0_ref, id1_ref, pht_ref, wph_ref, wbpe_ref, wtext_ref,
             out_ref, p2_ref):
    @pl.when(pl.program_id(0) == 0)
    def _():
        p2_ref[...] = _dot(_gelu(_dot(pht_ref[...], wph_ref[...])), wtext_ref[...])

    wbpe = wbpe_ref[...]
    acc = _gelu(_dot(g_ref[0], wbpe))
    for j in range(1, BPE_PER_WORD):
        acc = acc + _gelu(_dot(g_ref[j], wbpe))
    bm2 = _dot(acc * (1.0 / BPE_PER_WORD), wtext_ref[...])

    iota = lax.broadcasted_iota(jnp.int32, (WB, V_PH_PAD), 1)
    p2 = p2_ref[...]
    out = None
    for id_ref in (id0_ref, id1_ref):
        ids = jnp.broadcast_to(id_ref[...], (WB, V_PH_PAD))
        oh = (ids == iota).astype(jnp.float32)
        t = _gelu(_dot(oh, p2) + bm2)
        out = t if out is None else out + t
    out_ref[...] = out * (1.0 / PH_PER_WORD)


def _tc_fused(gathered, ids0, ids1, ph_table_pad, W_ph, W_bpe, W_text):
    full = lambda shape: pl.BlockSpec(shape, lambda i: tuple(0 for _ in shape))
    return pl.pallas_call(
        _tc_body,
        grid=(GRID,),
        in_specs=[
            pl.BlockSpec((BPE_PER_WORD, WB, D), lambda i: (0, i, 0)),
            pl.BlockSpec((WB, 1), lambda i: (i, 0)),
            pl.BlockSpec((WB, 1), lambda i: (i, 0)),
            full((V_PH_PAD, D)),
            full((D, D)),
            full((D, D)),
            full((D, D)),
        ],
        out_specs=pl.BlockSpec((WB, D), lambda i: (i, 0)),
        out_shape=jax.ShapeDtypeStruct((NB, D), jnp.float32),
        scratch_shapes=[pltpu.VMEM((V_PH_PAD, D), jnp.float32)],
    )(gathered, ids0, ids1, ph_table_pad, W_ph, W_bpe, W_text)


def kernel(bpe_ids, phoneme_ids, bpe_table, ph_table, W_bpe, W_ph, W_text):
    idx = bpe_ids.reshape(B, N_WORDS, BPE_PER_WORD).transpose(2, 0, 1)
    idx = idx.reshape(NW, NCHUNK, CHUNK)

    gathered = _sc_gather(idx, bpe_table)
    gathered = gathered.reshape(BPE_PER_WORD, NB, D)

    ph = phoneme_ids.reshape(B, N_WORDS, PH_PER_WORD)
    ids0 = ph[:, :, 0].reshape(NB, 1)
    ids1 = ph[:, :, 1].reshape(NB, 1)

    ph_table_pad = jnp.zeros((V_PH_PAD, D), jnp.float32).at[:ph_table.shape[0]].set(ph_table)

    return _tc_fused(gathered, ids0, ids1, ph_table_pad, W_ph, W_bpe, W_text)


# speedup vs baseline: 3.7564x; 1.0654x over previous
import functools

import jax
import jax.numpy as jnp
from jax import lax
from jax.experimental import pallas as pl
from jax.experimental.pallas import tpu as pltpu
from jax.experimental.pallas import tpu_sc as plsc

B = 8
L_BPE = 2048
BPE_PER_WORD = 4
PH_PER_WORD = 2
N_WORDS = L_BPE // BPE_PER_WORD
NB = B * N_WORDS
L_PH = N_WORDS * PH_PER_WORD
D = 256
V_PH_PAD = 128

NC = 2
NS = 16
NW = NC * NS
ROWS = BPE_PER_WORD * NB
ROWS_PER_W = ROWS // NW
CHUNK = 128
NCHUNK = ROWS_PER_W // CHUNK

WB = 1024
GRID = NB // WB


def _sc_gather(idx, table):
    mesh = plsc.VectorSubcoreMesh(core_axis_name="c", subcore_axis_name="s")

    @functools.partial(
        pl.kernel,
        mesh=mesh,
        out_type=jax.ShapeDtypeStruct((ROWS, D), jnp.float32),
        scratch_types=[
            pltpu.VMEM((NCHUNK, CHUNK), jnp.int32),
            pltpu.VMEM((CHUNK, D), jnp.float32),
            pltpu.VMEM((CHUNK, D), jnp.float32),
            pltpu.VMEM((CHUNK, D), jnp.float32),
            pltpu.SemaphoreType.DMA,
            pltpu.SemaphoreType.DMA,
            pltpu.SemaphoreType.DMA,
            pltpu.SemaphoreType.DMA,
            pltpu.SemaphoreType.DMA,
            pltpu.SemaphoreType.DMA,
        ],
    )
    def k(idx_hbm, table_hbm, out_hbm, idx_v, buf0, buf1, buf2,
          gs0, gs1, gs2, os0, os1, os2):
        wid = lax.axis_index("s") * NC + lax.axis_index("c")
        base = wid * ROWS_PER_W
        pltpu.sync_copy(idx_hbm.at[wid], idx_v)
        bufs = (buf0, buf1, buf2)
        gsems = (gs0, gs1, gs2)
        osems = (os0, os1, os2)
        gathers = [None] * NCHUNK
        outs = [None] * NCHUNK
        gathers[0] = pltpu.async_copy(table_hbm.at[idx_v.at[0]], bufs[0], gsems[0])
        if NCHUNK > 1:
            gathers[1] = pltpu.async_copy(table_hbm.at[idx_v.at[1]], bufs[1], gsems[1])
        for c in range(NCHUNK):
            gathers[c].wait()
            if c >= 1:
                outs[c - 1].wait()
            outs[c] = pltpu.async_copy(
                bufs[c % 3], out_hbm.at[pl.ds(base + c * CHUNK, CHUNK)], osems[c % 3]
            )
            if c + 2 < NCHUNK:
                gathers[c + 2] = pltpu.async_copy(
                    table_hbm.at[idx_v.at[c + 2]], bufs[(c + 2) % 3], gsems[(c + 2) % 3]
                )
        outs[NCHUNK - 1].wait()

    return k(idx, table)


_PREC = lax.Precision.DEFAULT


def _dot(a, b):
    return jnp.dot(a, b, preferred_element_type=jnp.float32, precision=_PREC)


_G1 = -2.0 * 0.7978845608028654 * 1.4426950408889634
_G3 = _G1 * 0.044715


def _gelu(x):
    x2 = x * x
    e = jnp.exp2(x * (_G1 + _G3 * x2))
    return x * lax.reciprocal(1.0 + e)


def _tc_body(g_ref, id0_ref, id1_ref, pht_ref, wph_ref, wbpe_ref, wtext_ref,
             out_ref, p2_ref):
    @pl.when(pl.program_id(0) == 0)
    def _():
        p2_ref[...] = _dot(_gelu(_dot(pht_ref[...], wph_ref[...])), wtext_ref[...])

    wbpe = wbpe_ref[...]
    acc = _gelu(_dot(g_ref[0], wbpe))
    for j in range(1, BPE_PER_WORD):
        acc = acc + _gelu(_dot(g_ref[j], wbpe))
    bm2 = _dot(acc * (1.0 / BPE_PER_WORD), wtext_ref[...])

    iota = lax.broadcasted_iota(jnp.int32, (WB, V_PH_PAD), 1)
    p2 = p2_ref[...]
    out = None
    for id_ref in (id0_ref, id1_ref):
        ids = jnp.broadcast_to(id_ref[...], (WB, V_PH_PAD))
        oh = (ids == iota).astype(jnp.float32)
        t = _gelu(_dot(oh, p2) + bm2)
        out = t if out is None else out + t
    out_ref[...] = out * (1.0 / PH_PER_WORD)


def _tc_fused(gathered, ids0, ids1, ph_table_pad, W_ph, W_bpe, W_text):
    full = lambda shape: pl.BlockSpec(shape, lambda i: tuple(0 for _ in shape))
    return pl.pallas_call(
        _tc_body,
        grid=(GRID,),
        in_specs=[
            pl.BlockSpec((BPE_PER_WORD, WB, D), lambda i: (0, i, 0)),
            pl.BlockSpec((WB, 1), lambda i: (i, 0)),
            pl.BlockSpec((WB, 1), lambda i: (i, 0)),
            full((V_PH_PAD, D)),
            full((D, D)),
            full((D, D)),
            full((D, D)),
        ],
        out_specs=pl.BlockSpec((WB, D), lambda i: (i, 0)),
        out_shape=jax.ShapeDtypeStruct((NB, D), jnp.float32),
        scratch_shapes=[pltpu.VMEM((V_PH_PAD, D), jnp.float32)],
    )(gathered, ids0, ids1, ph_table_pad, W_ph, W_bpe, W_text)


def kernel(bpe_ids, phoneme_ids, bpe_table, ph_table, W_bpe, W_ph, W_text):
    idx = bpe_ids.reshape(B, N_WORDS, BPE_PER_WORD).transpose(2, 0, 1)
    idx = idx.reshape(NW, NCHUNK, CHUNK)

    gathered = _sc_gather(idx, bpe_table)
    gathered = gathered.reshape(BPE_PER_WORD, NB, D)

    ph = phoneme_ids.reshape(B, N_WORDS, PH_PER_WORD)
    ids0 = ph[:, :, 0].reshape(NB, 1)
    ids1 = ph[:, :, 1].reshape(NB, 1)

    ph_table_pad = jnp.zeros((V_PH_PAD, D), jnp.float32).at[:ph_table.shape[0]].set(ph_table)

    return _tc_fused(gathered, ids0, ids1, ph_table_pad, W_ph, W_bpe, W_text)
